# baseline (speedup 1.0000x reference)
import functools

import jax
import jax.numpy as jnp
from jax import lax
from jax.experimental import pallas as pl
from jax.experimental.pallas import tpu as pltpu
from jax.experimental.pallas import tpu_sc as plsc

BATCH = 4096
SEQ = 50
DIM = 32
VOCAB = 1000000
NC = 2
NS = 16
NW = NC * NS
BW = BATCH // NW

NFULL = VOCAB // 128
VPAD = (NFULL + 1) * 128
BLKW = 128 * DIM
TAIL0 = NFULL * 128

_MESH = plsc.VectorSubcoreMesh(
    core_axis_name="c", subcore_axis_name="s", num_cores=NC, num_subcores=NS
)


@functools.partial(
    pl.kernel,
    mesh=_MESH,
    out_type=jax.ShapeDtypeStruct((VPAD * DIM,), jnp.float32),
    compiler_params=pltpu.CompilerParams(needs_layout_passes=False),
    scratch_types=[
        pltpu.VMEM((2, DIM, 129), jnp.float32),
        pltpu.VMEM((BLKW,), jnp.float32),
        pltpu.VMEM((BLKW,), jnp.float32),
        pltpu.VMEM((2048,), jnp.float32),
        pltpu.VMEM((128, 16), jnp.int32),
        pltpu.SemaphoreType.DMA,
        pltpu.SemaphoreType.DMA,
        pltpu.SemaphoreType.DMA,
        pltpu.SemaphoreType.DMA,
    ],
)
def _relayout_kernel(
    wt_hbm, tail_hbm, lin_hbm, gtile, tt0, tt1, tailv, ivc, r0, r1, w0, w1
):
    wid = lax.axis_index("s") * NC + lax.axis_index("c")
    rsems = (r0, r1)
    wsems = (w0, w1)
    tts = (tt0, tt1)
    lanes = lax.iota(jnp.int32, 16)

    for c in range(128):
        ivc[c, :] = jnp.full((16,), c, jnp.int32)

    def blk(t):
        return t * NW + wid

    def rd_desc(t, buf):
        cb = pl.multiple_of(blk(t) * 128, 128)
        return pltpu.make_async_copy(
            wt_hbm.at[:, pl.ds(cb, 128)],
            gtile.at[buf].at[:, pl.ds(0, 128)],
            rsems[buf],
        )

    def wr_desc(t, buf):
        r = pl.multiple_of(blk(t) * BLKW, 128)
        return pltpu.make_async_copy(
            tts[buf], lin_hbm.at[pl.ds(r, BLKW)], wsems[buf]
        )

    def transpose(buf):
        src = gtile.at[buf]
        dst = tts[buf]
        for h in range(2):
            rows_d = lanes + (h * 16)
            for c in range(128):
                cols = ivc[c, :]
                vals = plsc.load_gather(src, [rows_d, cols])
                dst[pl.ds(c * DIM + h * 16, 16)] = vals

    @pl.when(blk(0) < NFULL)
    def _():
        rd_desc(0, 0).start()

    def half(t, buf, nbuf):
        @pl.when(blk(t + 1) < NFULL)
        def _():
            rd_desc(t + 1, nbuf).start()

        @pl.when(blk(t) < NFULL)
        def _():
            rd_desc(t, buf).wait()

            @pl.when(t >= 2)
            def _():
                wr_desc(t - 2, buf).wait()

            transpose(buf)
            wr_desc(t, buf).start()

    def body(p, carry):
        t = p * 2
        half(t, 0, 1)
        half(t + 1, 1, 0)
        return carry

    lax.fori_loop(0, 123, body, 0)

    for tt in (242, 243, 244):
        @pl.when((blk(tt) < NFULL) & (blk(tt + 2) >= NFULL))
        def _(tt=tt):
            wr_desc(tt, tt % 2).wait()

    @pl.when(wid == 0)
    def _():
        pltpu.sync_copy(tail_hbm, tailv)
        pltpu.sync_copy(
            tailv, lin_hbm.at[pl.ds(TAIL0 * DIM, 2048)]
        )


@functools.partial(
    pl.kernel,
    mesh=_MESH,
    out_type=jax.ShapeDtypeStruct((SEQ, DIM, BATCH), jnp.float32),
    compiler_params=pltpu.CompilerParams(
        use_tc_tiling_on_sc=False, needs_layout_passes=False
    ),
    scratch_types=[
        pltpu.VMEM((SEQ, BW), jnp.int32),
        pltpu.VMEM((2, BW, DIM), jnp.float32),
        pltpu.VMEM((2, DIM, BW), jnp.float32),
        pltpu.VMEM((DIM, 16), jnp.int32),
        pltpu.VMEM((BW // 16, 16), jnp.int32),
        pltpu.SemaphoreType.DMA,
        pltpu.SemaphoreType.DMA,
        pltpu.SemaphoreType.DMA,
        pltpu.SemaphoreType.DMA,
    ],
)
def _gather_kernel(
    idx_hbm, table_hbm, out_hbm, idx_v, gbuf, tbuf, ivr, ivc, g0, g1, w0, w1
):
    wid = lax.axis_index("s") * NC + lax.axis_index("c")
    col0 = wid * BW
    gsems = (g0, g1)
    wsems = (w0, w1)
    lanes = lax.iota(jnp.int32, 16)

    for d0 in range(DIM):
        ivr[d0, :] = jnp.bitwise_and(lanes + d0, DIM - 1)
    for g in range(BW // 16):
        ivc[g, :] = lanes + (g * 16)

    pltpu.sync_copy(idx_hbm.at[:, pl.ds(col0, BW)], idx_v)

    def g_desc(s, buf):
        return pltpu.make_async_copy(
            table_hbm.at[idx_v.at[s]], gbuf.at[buf], gsems[buf]
        )

    def w_desc(s, buf):
        return pltpu.make_async_copy(
            tbuf.at[buf], out_hbm.at[s].at[:, pl.ds(col0, BW)], wsems[buf]
        )

    def transpose(buf):
        src = gbuf.at[buf]
        dst = tbuf.at[buf]
        for g in range(BW // 16):
            cols2 = ivc[g, :]
            for d0 in range(DIM):
                rows2 = ivr[d0, :]
                vals = plsc.load_gather(src, [cols2, rows2])
                plsc.store_scatter(dst, [rows2, cols2], vals)

    g_desc(0, 0).start()

    def half(s, buf, nbuf):
        @pl.when(s + 1 < SEQ)
        def _():
            g_desc(s + 1, nbuf).start()

        g_desc(s, buf).wait()

        @pl.when(s >= 2)
        def _():
            w_desc(s - 2, buf).wait()

        transpose(buf)
        w_desc(s, buf).start()

    def body(p, carry):
        s = p * 2
        half(s, 0, 1)
        half(s + 1, 1, 0)
        return carry

    lax.fori_loop(0, SEQ // 2, body, 0)
    w_desc(SEQ - 2, 0).wait()
    w_desc(SEQ - 1, 1).wait()


def kernel(model_input, weight):
    w_t = weight.T
    w_tail = lax.slice(weight, (TAIL0, 0), (VOCAB, DIM)).reshape(2048)
    w_lin = _relayout_kernel(w_t, w_tail)
    table = w_lin.reshape(VPAD, DIM)
    idx_t = model_input.T.astype(jnp.int32)
    out_t = _gather_kernel(idx_t, table)
    return out_t.transpose(2, 0, 1)
